# baseline (speedup 1.0000x reference)
import functools

import jax
import jax.numpy as jnp
from jax.experimental import pallas as pl
from jax.experimental.pallas import tpu as pltpu

N, D, H, K = 10000, 128, 64, 16
BLOCK_N = 400
EPI_CHUNK = 2000


def _xw_kernel(x_ref, w1_ref, bn_ref, o_ref):
    gamma, rvar = bn_ref[0, :], bn_ref[3, :]
    scale = gamma * jax.lax.rsqrt(rvar + 1e-5)
    o_ref[...] = jnp.dot(x_ref[...], w1_ref[...],
                         preferred_element_type=jnp.float32) * scale


def _fused_kernel(xw_in_ref, a_ref, h_ref, bn_ref, wih_ref, whh_ref,
                  bias_ref, out_y_ref, out_h_ref, acc_ref):
    i = pl.program_id(0)
    nsteps = pl.num_programs(0)
    xw_ref = xw_in_ref

    rows = pl.ds(i * BLOCK_N, BLOCK_N)
    acc_ref[rows, :] = jnp.dot(a_ref[...], xw_ref[...],
                               preferred_element_type=jnp.float32)

    @pl.when(i == nsteps - 1)
    def _epilogue():
        gamma, beta, rmean, rvar, b1 = (bn_ref[0, :], bn_ref[1, :],
                                        bn_ref[2, :], bn_ref[3, :], bn_ref[4, :])
        scale = gamma * jax.lax.rsqrt(rvar + 1e-5)
        shift = (b1 - rmean) * scale + beta
        alpha = 1.6732632423543772
        lam = 1.0507009873554805
        wih = wih_ref[...]
        whh = whh_ref[...]
        bih = bias_ref[0, :]
        bhh = bias_ref[1, :]

        def _chunk(c, carry):
            rows = pl.ds(c * EPI_CHUNK, EPI_CHUNK)
            y = acc_ref[rows, :] + shift
            y = lam * jnp.where(y > 0, y, alpha * (jnp.exp(y) - 1.0))
            h = h_ref[rows, :]
            gi = jnp.dot(y, wih, preferred_element_type=jnp.float32) + bih
            gh = jnp.dot(h, whh, preferred_element_type=jnp.float32) + bhh
            r = jax.nn.sigmoid(gi[:, 0:K] + gh[:, 0:K])
            z = jax.nn.sigmoid(gi[:, K:2 * K] + gh[:, K:2 * K])
            n = jnp.tanh(gi[:, 2 * K:3 * K] + r * gh[:, 2 * K:3 * K])
            out_h_ref[rows, :] = n + z * (h - n)
            out_y_ref[rows, :] = y
            return carry

        jax.lax.fori_loop(0, N // EPI_CHUNK, _chunk, 0)


@functools.partial(jax.jit, static_argnames=("interpret",))
def _run(x, a, h0, W1, b1, gamma, beta, rmean, rvar, Wih, Whh, bih, bhh,
         interpret=False):
    bn = jnp.stack([gamma, beta, rmean, rvar, b1], axis=0)
    bias = jnp.stack([bih, bhh], axis=0)

    grid = (N // BLOCK_N,)
    row = lambda i: (i, 0)
    rep = lambda i: (0, 0)
    xw = pl.pallas_call(
        _xw_kernel,
        out_shape=jax.ShapeDtypeStruct((N, H), jnp.float32),
        interpret=interpret,
    )(x, W1, bn)
    out_y, out_h = pl.pallas_call(
        _fused_kernel,
        grid=grid,
        in_specs=[
            pl.BlockSpec((N, H), rep),
            pl.BlockSpec((BLOCK_N, N), row),
            pl.BlockSpec((N, K), rep),
            pl.BlockSpec((5, H), rep),
            pl.BlockSpec((H, 3 * K), rep),
            pl.BlockSpec((K, 3 * K), rep),
            pl.BlockSpec((2, 3 * K), rep),
        ],
        out_specs=[
            pl.BlockSpec((N, H), rep),
            pl.BlockSpec((N, K), rep),
        ],
        out_shape=[
            jax.ShapeDtypeStruct((N, H), jnp.float32),
            jax.ShapeDtypeStruct((N, K), jnp.float32),
        ],
        scratch_shapes=[
            pltpu.VMEM((N, H), jnp.float32),
        ],
        compiler_params=pltpu.CompilerParams(
            dimension_semantics=("arbitrary",),
        ),
        interpret=interpret,
    )(xw, a, h0, bn, Wih.T, Whh.T, bias)
    return out_y, out_h


def kernel(features_list, norm_adjacency_list, adjacency_list,
           init_assignments, W1, b1, gamma, beta, rmean, rvar,
           Wih, Whh, bih, bhh, interpret=False):
    x = features_list[0]
    a = norm_adjacency_list[0]
    out_y, out_h = _run(x, a, init_assignments, W1, b1, gamma, beta,
                        rmean, rvar, Wih, Whh, bih, bhh,
                        interpret=interpret)
    return (out_h[None], out_y[None])


# speedup vs baseline: 1.0084x; 1.0084x over previous
import functools

import jax
import jax.numpy as jnp
from jax.experimental import pallas as pl
from jax.experimental.pallas import tpu as pltpu

N, D, H, K = 10000, 128, 64, 16
BLOCK_N = 400
EPI_CHUNK = 2000


def _xw_kernel(x_ref, w1_ref, bn_ref, o_ref):
    gamma, rvar = bn_ref[0, :], bn_ref[3, :]
    scale = gamma * jax.lax.rsqrt(rvar + 1e-5)
    o_ref[...] = jnp.dot(x_ref[...], w1_ref[...],
                         preferred_element_type=jnp.float32) * scale


def _fused_kernel(xw_in_ref, a_ref, h_ref, bn_ref, wih_ref, whh_ref,
                  bias_ref, out_y_ref, out_h_ref, acc_ref, hs_ref,
                  sem_y, sem_h):
    i = pl.program_id(0)
    nsteps = pl.num_programs(0)
    xw_ref = xw_in_ref

    rows = pl.ds(i * BLOCK_N, BLOCK_N)
    acc_ref[rows, :] = jnp.dot(a_ref[...], xw_ref[...],
                               preferred_element_type=jnp.float32)

    @pl.when(i == nsteps - 1)
    def _epilogue():
        gamma, beta, rmean, rvar, b1 = (bn_ref[0, :], bn_ref[1, :],
                                        bn_ref[2, :], bn_ref[3, :], bn_ref[4, :])
        scale = gamma * jax.lax.rsqrt(rvar + 1e-5)
        shift = (b1 - rmean) * scale + beta
        alpha = 1.6732632423543772
        lam = 1.0507009873554805
        wih = wih_ref[...]
        whh = whh_ref[...]
        bih = bias_ref[0, :]
        bhh = bias_ref[1, :]

        def _chunk(c, carry):
            rows = pl.ds(c * EPI_CHUNK, EPI_CHUNK)
            y = acc_ref[rows, :] + shift
            y = lam * jnp.where(y > 0, y, alpha * (jnp.exp(y) - 1.0))
            h = h_ref[rows, :]
            gi = jnp.dot(y, wih, preferred_element_type=jnp.float32) + bih
            gh = jnp.dot(h, whh, preferred_element_type=jnp.float32) + bhh
            r = jax.nn.sigmoid(gi[:, 0:K] + gh[:, 0:K])
            z = jax.nn.sigmoid(gi[:, K:2 * K] + gh[:, K:2 * K])
            n = jnp.tanh(gi[:, 2 * K:3 * K] + r * gh[:, 2 * K:3 * K])
            hs_ref[rows, :] = n + z * (h - n)
            acc_ref[rows, :] = y
            return carry

        jax.lax.fori_loop(0, N // EPI_CHUNK, _chunk, 0)
        cp_y = pltpu.make_async_copy(acc_ref, out_y_ref, sem_y)
        cp_h = pltpu.make_async_copy(hs_ref, out_h_ref, sem_h)
        cp_y.start()
        cp_h.start()
        cp_y.wait()
        cp_h.wait()


@functools.partial(jax.jit, static_argnames=("interpret",))
def _run(x, a, h0, W1, b1, gamma, beta, rmean, rvar, Wih, Whh, bih, bhh,
         interpret=False):
    bn = jnp.stack([gamma, beta, rmean, rvar, b1], axis=0)
    bias = jnp.stack([bih, bhh], axis=0)

    grid = (N // BLOCK_N,)
    row = lambda i: (i, 0)
    rep = lambda i: (0, 0)
    xw = pl.pallas_call(
        _xw_kernel,
        out_shape=jax.ShapeDtypeStruct((N, H), jnp.float32),
        interpret=interpret,
    )(x, W1, bn)
    out_y, out_h = pl.pallas_call(
        _fused_kernel,
        grid=grid,
        in_specs=[
            pl.BlockSpec((N, H), rep),
            pl.BlockSpec((BLOCK_N, N), row),
            pl.BlockSpec((N, K), rep),
            pl.BlockSpec((5, H), rep),
            pl.BlockSpec((H, 3 * K), rep),
            pl.BlockSpec((K, 3 * K), rep),
            pl.BlockSpec((2, 3 * K), rep),
        ],
        out_specs=[
            pl.BlockSpec(memory_space=pltpu.MemorySpace.HBM),
            pl.BlockSpec(memory_space=pltpu.MemorySpace.HBM),
        ],
        out_shape=[
            jax.ShapeDtypeStruct((N, H), jnp.float32),
            jax.ShapeDtypeStruct((N, K), jnp.float32),
        ],
        scratch_shapes=[
            pltpu.VMEM((N, H), jnp.float32),
            pltpu.VMEM((N, K), jnp.float32),
            pltpu.SemaphoreType.DMA,
            pltpu.SemaphoreType.DMA,
        ],
        compiler_params=pltpu.CompilerParams(
            dimension_semantics=("arbitrary",),
        ),
        interpret=interpret,
    )(xw, a, h0, bn, Wih.T, Whh.T, bias)
    return out_y, out_h


def kernel(features_list, norm_adjacency_list, adjacency_list,
           init_assignments, W1, b1, gamma, beta, rmean, rvar,
           Wih, Whh, bih, bhh, interpret=False):
    x = features_list[0]
    a = norm_adjacency_list[0]
    out_y, out_h = _run(x, a, init_assignments, W1, b1, gamma, beta,
                        rmean, rvar, Wih, Whh, bih, bhh,
                        interpret=interpret)
    return (out_h[None], out_y[None])


# speedup vs baseline: 1.0611x; 1.0523x over previous
import functools

import jax
import jax.numpy as jnp
from jax.experimental import pallas as pl
from jax.experimental.pallas import tpu as pltpu

N, D, H, K = 10000, 128, 64, 16
BLOCK_N = 400
EPI_CHUNK = 2000


def _xw_kernel(x_ref, w1_ref, bn_ref, o_ref):
    gamma, rvar = bn_ref[0, :], bn_ref[3, :]
    scale = gamma * jax.lax.rsqrt(rvar + 1e-5)
    o_ref[...] = jnp.dot(x_ref[...], w1_ref[...],
                         preferred_element_type=jnp.float32) * scale


def _fused_kernel(xw_in_ref, a_ref, h_ref, bn_ref, wih_ref, whh_ref,
                  bias_ref, out_y_ref, out_h_ref, acc_ref, hs_ref,
                  sem_y, sem_h):
    i = pl.program_id(0)
    nsteps = pl.num_programs(0)
    xw_ref = xw_in_ref

    gamma, beta, rmean, rvar, b1 = (bn_ref[0, :], bn_ref[1, :],
                                    bn_ref[2, :], bn_ref[3, :], bn_ref[4, :])
    scale = gamma * jax.lax.rsqrt(rvar + 1e-5)
    shift = (b1 - rmean) * scale + beta
    alpha = 1.6732632423543772
    lam = 1.0507009873554805

    rows = pl.ds(i * BLOCK_N, BLOCK_N)
    y = jnp.dot(a_ref[...], xw_ref[...],
                preferred_element_type=jnp.float32) + shift
    y = lam * jnp.where(y > 0, y, alpha * (jnp.exp(y) - 1.0))
    h = h_ref[rows, :]
    gi = jnp.dot(y, wih_ref[...], preferred_element_type=jnp.float32) + bias_ref[0, :]
    gh = jnp.dot(h, whh_ref[...], preferred_element_type=jnp.float32) + bias_ref[1, :]
    r = jax.nn.sigmoid(gi[:, 0:K] + gh[:, 0:K])
    z = jax.nn.sigmoid(gi[:, K:2 * K] + gh[:, K:2 * K])
    n = jnp.tanh(gi[:, 2 * K:3 * K] + r * gh[:, 2 * K:3 * K])
    hs_ref[rows, :] = n + z * (h - n)
    acc_ref[rows, :] = y

    @pl.when(i == nsteps - 1)
    def _epilogue():
        cp_y = pltpu.make_async_copy(acc_ref, out_y_ref, sem_y)
        cp_h = pltpu.make_async_copy(hs_ref, out_h_ref, sem_h)
        cp_y.start()
        cp_h.start()
        cp_y.wait()
        cp_h.wait()


@functools.partial(jax.jit, static_argnames=("interpret",))
def _run(x, a, h0, W1, b1, gamma, beta, rmean, rvar, Wih, Whh, bih, bhh,
         interpret=False):
    bn = jnp.stack([gamma, beta, rmean, rvar, b1], axis=0)
    bias = jnp.stack([bih, bhh], axis=0)

    grid = (N // BLOCK_N,)
    row = lambda i: (i, 0)
    rep = lambda i: (0, 0)
    xw = pl.pallas_call(
        _xw_kernel,
        out_shape=jax.ShapeDtypeStruct((N, H), jnp.float32),
        interpret=interpret,
    )(x, W1, bn)
    out_y, out_h = pl.pallas_call(
        _fused_kernel,
        grid=grid,
        in_specs=[
            pl.BlockSpec((N, H), rep),
            pl.BlockSpec((BLOCK_N, N), row),
            pl.BlockSpec((N, K), rep),
            pl.BlockSpec((5, H), rep),
            pl.BlockSpec((H, 3 * K), rep),
            pl.BlockSpec((K, 3 * K), rep),
            pl.BlockSpec((2, 3 * K), rep),
        ],
        out_specs=[
            pl.BlockSpec(memory_space=pltpu.MemorySpace.HBM),
            pl.BlockSpec(memory_space=pltpu.MemorySpace.HBM),
        ],
        out_shape=[
            jax.ShapeDtypeStruct((N, H), jnp.float32),
            jax.ShapeDtypeStruct((N, K), jnp.float32),
        ],
        scratch_shapes=[
            pltpu.VMEM((N, H), jnp.float32),
            pltpu.VMEM((N, K), jnp.float32),
            pltpu.SemaphoreType.DMA,
            pltpu.SemaphoreType.DMA,
        ],
        compiler_params=pltpu.CompilerParams(
            dimension_semantics=("arbitrary",),
        ),
        interpret=interpret,
    )(xw, a, h0, bn, Wih.T, Whh.T, bias)
    return out_y, out_h


def kernel(features_list, norm_adjacency_list, adjacency_list,
           init_assignments, W1, b1, gamma, beta, rmean, rvar,
           Wih, Whh, bih, bhh, interpret=False):
    x = features_list[0]
    a = norm_adjacency_list[0]
    out_y, out_h = _run(x, a, init_assignments, W1, b1, gamma, beta,
                        rmean, rvar, Wih, Whh, bih, bhh,
                        interpret=interpret)
    return (out_h[None], out_y[None])
